# baseline (speedup 1.0000x reference)
import functools

import jax
import jax.numpy as jnp
from jax import lax
from jax.experimental import pallas as pl
from jax.experimental.pallas import tpu as pltpu
from jax.experimental.pallas import tpu_sc as plsc

N = 10000
D = 128
E = 320000
NC = 2
NS = 16
NW = NC * NS
CHUNK = 128
E_TOT = E + N
CPT = 84
E_PAD = NW * CPT * CHUNK
DUMP = N
HOP_CHUNK = 64
HOP_CPT = E_PAD // (NW * HOP_CHUNK)
PASS_SZ = (88, 80)
N_PAD = 10368
RPT = N_PAD // NS

_MESH = plsc.VectorSubcoreMesh(core_axis_name="c", subcore_axis_name="s")


def _hist_body(col_hbm, z_hbm, ones_hbm, out, dacc, idx_c, ones_v, sem):
    c = lax.axis_index("c")
    s = lax.axis_index("s")
    w = c * NS + s
    pltpu.sync_copy(col_hbm.at[w], idx_c)
    pltpu.sync_copy(ones_hbm, ones_v)
    r0 = s * RPT
    pltpu.sync_copy(z_hbm.at[pl.ds(r0, RPT)], dacc.at[pl.ds(r0, RPT)])
    plsc.subcore_barrier()

    def body(j, carry):
        pltpu.sync_copy(ones_v, dacc.at[idx_c.at[j]], add=True)
        return carry

    lax.fori_loop(0, CPT, body, 0)
    plsc.subcore_barrier()
    pltpu.sync_copy(dacc.at[pl.ds(r0, RPT)], out.at[c, pl.ds(r0, RPT)])


_hist = pl.kernel(
    _hist_body,
    out_type=jax.ShapeDtypeStruct((NC, N_PAD, D), jnp.float32),
    mesh=_MESH,
    scratch_types=[
        pltpu.VMEM_SHARED((N_PAD, D), jnp.float32),
        pltpu.VMEM((CPT, CHUNK), jnp.int32),
        pltpu.VMEM((CHUNK, D), jnp.float32),
        pltpu.SemaphoreType.DMA,
    ],
)


def _hop_body(g_hbm, row_hbm, col_hbm, z_hbm, out, acc, idx_r, idx_c,
              buf0, buf1, sem0, sem1):
    c = lax.axis_index("c")
    s = lax.axis_index("s")
    w = c * NS + s
    r0 = s * RPT
    pltpu.sync_copy(z_hbm.at[pl.ds(r0, RPT)], acc.at[pl.ds(r0, RPT)])
    plsc.subcore_barrier()

    def make_body(sz):
        def body(k, carry):
            pltpu.async_copy(g_hbm.at[idx_r.at[k]], buf0, sem0).wait()
            pltpu.sync_copy(buf0, acc.at[idx_c.at[k]], add=True)
            return carry
        return body

    base = 0
    for sz in PASS_SZ:
        pltpu.sync_copy(row_hbm.at[w, pl.ds(base, sz)], idx_r.at[pl.ds(0, sz)])
        pltpu.sync_copy(col_hbm.at[w, pl.ds(base, sz)], idx_c.at[pl.ds(0, sz)])
        lax.fori_loop(0, sz, make_body(sz), 0)
        base += sz

    plsc.subcore_barrier()
    pltpu.sync_copy(acc.at[pl.ds(r0, RPT)], out.at[c, pl.ds(r0, RPT)])


_hop = pl.kernel(
    _hop_body,
    out_type=jax.ShapeDtypeStruct((NC, N_PAD, D), jnp.float32),
    mesh=_MESH,
    scratch_types=[
        pltpu.VMEM_SHARED((N_PAD, D), jnp.float32),
        pltpu.VMEM((PASS_SZ[0], HOP_CHUNK), jnp.int32),
        pltpu.VMEM((PASS_SZ[0], HOP_CHUNK), jnp.int32),
        pltpu.VMEM((HOP_CHUNK, D), jnp.float32),
        pltpu.VMEM((HOP_CHUNK, D), jnp.float32),
        pltpu.SemaphoreType.DMA,
        pltpu.SemaphoreType.DMA,
    ],
)


def _dinv_block(dg_ref):
    deg = dg_ref[0, :, 0] + dg_ref[1, :, 0]
    return jnp.where(deg > 0, 1.0 / jnp.sqrt(deg), 0.0)


def _scale0_body(dg_ref, x_ref, o_ref):
    dinv = _dinv_block(dg_ref)
    o_ref[...] = x_ref[...] * dinv[:, None]


def _mid_body(dg_ref, p_ref, o_ref):
    dinv = _dinv_block(dg_ref)
    o_ref[...] = (p_ref[0] + p_ref[1]) * (dinv * dinv)[:, None]


def _final_body(dg_ref, p_ref, w_ref, b_ref, o_ref):
    dinv = _dinv_block(dg_ref)
    h = (p_ref[0] + p_ref[1]) * dinv[:, None]
    z = jnp.dot(h, w_ref[...], preferred_element_type=jnp.float32) + b_ref[...]
    m = jnp.max(z, axis=1, keepdims=True)
    zz = z - m
    lse = jnp.log(jnp.sum(jnp.exp(zz), axis=1, keepdims=True))
    o_ref[...] = zz - lse


_BR = 576
_deg_spec = pl.BlockSpec((NC, _BR, D), lambda i: (0, i, 0))
_row_spec = pl.BlockSpec((_BR, D), lambda i: (i, 0))
_p_spec = pl.BlockSpec((NC, _BR, D), lambda i: (0, i, 0))

_scale0 = pl.pallas_call(
    _scale0_body,
    grid=(N_PAD // _BR,),
    in_specs=[_deg_spec, _row_spec],
    out_specs=_row_spec,
    out_shape=jax.ShapeDtypeStruct((N_PAD, D), jnp.float32),
)

_mid = pl.pallas_call(
    _mid_body,
    grid=(N_PAD // _BR,),
    in_specs=[_deg_spec, _p_spec],
    out_specs=_row_spec,
    out_shape=jax.ShapeDtypeStruct((N_PAD, D), jnp.float32),
)

_BF = 400
_final = pl.pallas_call(
    _final_body,
    grid=(N // _BF,),
    in_specs=[
        pl.BlockSpec((NC, _BF, D), lambda i: (0, i, 0)),
        pl.BlockSpec((NC, _BF, D), lambda i: (0, i, 0)),
        pl.BlockSpec((D, D), lambda i: (0, 0)),
        pl.BlockSpec((1, D), lambda i: (0, 0)),
    ],
    out_specs=pl.BlockSpec((_BF, D), lambda i: (i, 0)),
    out_shape=jax.ShapeDtypeStruct((N, D), jnp.float32),
)


def kernel(x, edge_index, W, b):
    loops = jnp.arange(N, dtype=jnp.int32)
    pad = jnp.full((E_PAD - E_TOT,), DUMP, dtype=jnp.int32)
    row = jnp.concatenate([edge_index[0], loops, pad]).reshape(NW, CPT, CHUNK)
    col = jnp.concatenate([edge_index[1], loops, pad]).reshape(NW, CPT, CHUNK)
    x_pad = jnp.pad(x, ((0, N_PAD - N), (0, 0)))
    z128 = jnp.zeros((N_PAD, D), jnp.float32)
    ones128 = jnp.ones((CHUNK, D), jnp.float32)

    row_h = row.reshape(NW, HOP_CPT, HOP_CHUNK)
    col_h = col.reshape(NW, HOP_CPT, HOP_CHUNK)
    dg = _hist(col, z128, ones128)
    g0 = _scale0(dg, x_pad)
    p1 = _hop(g0, row_h, col_h, z128)
    g1 = _mid(dg, p1)
    p2 = _hop(g1, row_h, col_h, z128)
    return _final(dg, p2, W, b.reshape(1, D))


# speedup vs baseline: 5.2689x; 5.2689x over previous
import jax
import jax.numpy as jnp
from jax import lax
from jax.experimental import pallas as pl
from jax.experimental.pallas import tpu as pltpu
from jax.experimental.pallas import tpu_sc as plsc

N = 10000
D = 128
E = 320000
NC = 2
NS = 16
NW = NC * NS
CHUNK = 128
E_TOT = E + N
CPT = 88
E_PAD = NW * CPT * CHUNK
PASS_SZ = (48, 40)
N_PAD = 10368
ACC_R = 10112
ART = ACC_R // NS
RPT = N_PAD // NS

_MESH = plsc.VectorSubcoreMesh(core_axis_name="c", subcore_axis_name="s")


def _hist_body(col_hbm, z_hbm, ones_hbm, out, dacc, idx_c, ones_v, sem):
    c = lax.axis_index("c")
    s = lax.axis_index("s")
    w = c * NS + s
    pltpu.sync_copy(col_hbm.at[w], idx_c)
    pltpu.sync_copy(ones_hbm, ones_v)
    r0 = s * RPT
    pltpu.sync_copy(z_hbm.at[pl.ds(r0, RPT)], dacc.at[pl.ds(r0, RPT)])
    plsc.subcore_barrier()

    def body(j, carry):
        pltpu.sync_copy(ones_v, dacc.at[idx_c.at[j]], add=True)
        return carry

    lax.fori_loop(0, CPT, body, 0)
    plsc.subcore_barrier()
    pltpu.sync_copy(dacc.at[pl.ds(r0, RPT)], out.at[c, pl.ds(r0, RPT)])


_hist = pl.kernel(
    _hist_body,
    out_type=jax.ShapeDtypeStruct((NC, N_PAD, D), jnp.float32),
    mesh=_MESH,
    scratch_types=[
        pltpu.VMEM_SHARED((N_PAD, D), jnp.float32),
        pltpu.VMEM((CPT, CHUNK), jnp.int32),
        pltpu.VMEM((CHUNK, D), jnp.float32),
        pltpu.SemaphoreType.DMA,
    ],
)


def _hop_body(g_hbm, row_hbm, col_hbm, z_hbm, out, acc, idx_r, idx_c,
              buf0, buf1, sem0, sem1):
    c = lax.axis_index("c")
    s = lax.axis_index("s")
    w = c * NS + s
    r0 = s * ART
    pltpu.sync_copy(z_hbm.at[pl.ds(r0, ART)], acc.at[pl.ds(r0, ART)])
    plsc.subcore_barrier()

    def make_body(sz):
        def body(k, carry):
            j0 = 2 * k
            j1 = j0 + 1
            j2 = j0 + 2

            pltpu.async_copy(g_hbm.at[idx_r.at[j1]], buf1, sem1)
            pltpu.make_async_copy(g_hbm.at[idx_r.at[j0]], buf0, sem0).wait()
            pltpu.sync_copy(buf0, acc.at[idx_c.at[j0]], add=True)

            @pl.when(j2 < sz)
            def _():
                pltpu.async_copy(g_hbm.at[idx_r.at[j2]], buf0, sem0)

            pltpu.make_async_copy(g_hbm.at[idx_r.at[j1]], buf1, sem1).wait()
            pltpu.sync_copy(buf1, acc.at[idx_c.at[j1]], add=True)
            return carry
        return body

    base = 0
    for sz in PASS_SZ:
        pltpu.sync_copy(row_hbm.at[w, pl.ds(base, sz)], idx_r.at[pl.ds(0, sz)])
        pltpu.sync_copy(col_hbm.at[w, pl.ds(base, sz)], idx_c.at[pl.ds(0, sz)])
        pltpu.async_copy(g_hbm.at[idx_r.at[0]], buf0, sem0)
        lax.fori_loop(0, sz // 2, make_body(sz), 0)
        base += sz

    plsc.subcore_barrier()
    pltpu.sync_copy(acc.at[pl.ds(r0, ART)], out.at[c, pl.ds(r0, ART)])


_hop = pl.kernel(
    _hop_body,
    out_type=jax.ShapeDtypeStruct((NC, N_PAD, D), jnp.float32),
    mesh=_MESH,
    scratch_types=[
        pltpu.VMEM_SHARED((ACC_R, D), jnp.float32),
        pltpu.VMEM((PASS_SZ[0], CHUNK), jnp.int32),
        pltpu.VMEM((PASS_SZ[0], CHUNK), jnp.int32),
        pltpu.VMEM((CHUNK, D), jnp.float32),
        pltpu.VMEM((CHUNK, D), jnp.float32),
        pltpu.SemaphoreType.DMA,
        pltpu.SemaphoreType.DMA,
    ],
)


def _dinv_block(dg_ref):
    deg = dg_ref[0, :, 0] + dg_ref[1, :, 0]
    return jnp.where(deg > 0, 1.0 / jnp.sqrt(deg), 0.0)


def _scale0_body(dg_ref, x_ref, o_ref):
    dinv = _dinv_block(dg_ref)
    o_ref[...] = x_ref[...] * dinv[:, None]


def _mid_body(dg_ref, p_ref, o_ref):
    dinv = _dinv_block(dg_ref)
    o_ref[...] = (p_ref[0] + p_ref[1]) * (dinv * dinv)[:, None]


def _final_body(dg_ref, p_ref, w_ref, b_ref, o_ref):
    dinv = _dinv_block(dg_ref)
    h = (p_ref[0] + p_ref[1]) * dinv[:, None]
    z = jnp.dot(h, w_ref[...], preferred_element_type=jnp.float32) + b_ref[...]
    m = jnp.max(z, axis=1, keepdims=True)
    zz = z - m
    lse = jnp.log(jnp.sum(jnp.exp(zz), axis=1, keepdims=True))
    o_ref[...] = zz - lse


_BR = 576
_deg_spec = pl.BlockSpec((NC, _BR, D), lambda i: (0, i, 0))
_row_spec = pl.BlockSpec((_BR, D), lambda i: (i, 0))
_p_spec = pl.BlockSpec((NC, _BR, D), lambda i: (0, i, 0))

_scale0 = pl.pallas_call(
    _scale0_body,
    grid=(N_PAD // _BR,),
    in_specs=[_deg_spec, _row_spec],
    out_specs=_row_spec,
    out_shape=jax.ShapeDtypeStruct((N_PAD, D), jnp.float32),
)

_mid = pl.pallas_call(
    _mid_body,
    grid=(N_PAD // _BR,),
    in_specs=[_deg_spec, _p_spec],
    out_specs=_row_spec,
    out_shape=jax.ShapeDtypeStruct((N_PAD, D), jnp.float32),
)

_BF = 400
_final = pl.pallas_call(
    _final_body,
    grid=(N // _BF,),
    in_specs=[
        pl.BlockSpec((NC, _BF, D), lambda i: (0, i, 0)),
        pl.BlockSpec((NC, _BF, D), lambda i: (0, i, 0)),
        pl.BlockSpec((D, D), lambda i: (0, 0)),
        pl.BlockSpec((1, D), lambda i: (0, 0)),
    ],
    out_specs=pl.BlockSpec((_BF, D), lambda i: (i, 0)),
    out_shape=jax.ShapeDtypeStruct((N, D), jnp.float32),
)


def kernel(x, edge_index, W, b):
    loops = jnp.arange(N, dtype=jnp.int32)
    padn = E_PAD - E_TOT
    pad_seq = jnp.arange(padn, dtype=jnp.int32)
    row_pad = N + pad_seq % (ACC_R - N)
    colh_pad = pad_seq % ACC_R
    cold_pad = N + pad_seq % (N_PAD - N)

    row3 = jnp.concatenate([edge_index[0], loops, row_pad]).reshape(NW, CPT, CHUNK)
    colh3 = jnp.concatenate([edge_index[1], loops, colh_pad]).reshape(NW, CPT, CHUNK)
    cold3 = jnp.concatenate([edge_index[1], loops, cold_pad]).reshape(NW, CPT, CHUNK)
    x_pad = jnp.pad(x, ((0, N_PAD - N), (0, 0)))
    z128 = jnp.zeros((N_PAD, D), jnp.float32)
    ones128 = jnp.ones((CHUNK, D), jnp.float32)

    dg = _hist(cold3, z128, ones128)
    g0 = _scale0(dg, x_pad)
    p1 = _hop(g0, row3, colh3, z128)
    g1 = _mid(dg, p1)
    p2 = _hop(g1, row3, colh3, z128)
    return _final(dg, p2, W, b.reshape(1, D))


# speedup vs baseline: 5.6046x; 1.0637x over previous
import jax
import jax.numpy as jnp
from jax import lax
from jax.experimental import pallas as pl
from jax.experimental.pallas import tpu as pltpu
from jax.experimental.pallas import tpu_sc as plsc

N = 10000
D = 128
E = 320000
NC = 2
NS = 16
NW = NC * NS
CHUNK = 128
CPT = 80
E_PAD = NW * CPT * CHUNK
PASS_SZ = (40, 40)
N_PAD = 10368
ACC_R = 10112
ART = ACC_R // NS
RPT = N_PAD // NS

_MESH = plsc.VectorSubcoreMesh(core_axis_name="c", subcore_axis_name="s")


def _hist_body(col_hbm, z_hbm, ones_hbm, out, dacc, idx_c, ones_v, sem):
    c = lax.axis_index("c")
    s = lax.axis_index("s")
    w = c * NS + s
    pltpu.sync_copy(col_hbm.at[w], idx_c)
    pltpu.sync_copy(ones_hbm, ones_v)
    r0 = s * RPT
    pltpu.sync_copy(z_hbm.at[pl.ds(r0, RPT)], dacc.at[pl.ds(r0, RPT)])
    plsc.subcore_barrier()

    def body(j, carry):
        pltpu.sync_copy(ones_v, dacc.at[idx_c.at[j]], add=True)
        return carry

    lax.fori_loop(0, CPT, body, 0)
    plsc.subcore_barrier()
    pltpu.sync_copy(dacc.at[pl.ds(r0, RPT)], out.at[c, pl.ds(r0, RPT)])


_hist = pl.kernel(
    _hist_body,
    out_type=jax.ShapeDtypeStruct((NC, N_PAD, D), jnp.float32),
    mesh=_MESH,
    scratch_types=[
        pltpu.VMEM_SHARED((N_PAD, D), jnp.float32),
        pltpu.VMEM((CPT, CHUNK), jnp.int32),
        pltpu.VMEM((CHUNK, D), jnp.float32),
        pltpu.SemaphoreType.DMA,
    ],
)


def _hop_body(g_hbm, row_hbm, col_hbm, gh_hbm, out, acc, idx_r, idx_c,
              buf0, buf1, sem0, sem1):
    c = lax.axis_index("c")
    s = lax.axis_index("s")
    w = c * NS + s
    r0 = s * ART
    pltpu.sync_copy(gh_hbm.at[pl.ds(r0, ART)], acc.at[pl.ds(r0, ART)])
    plsc.subcore_barrier()

    def make_body(sz):
        def body(k, carry):
            j0 = 2 * k
            j1 = j0 + 1
            j2 = j0 + 2

            pltpu.async_copy(g_hbm.at[idx_r.at[j1]], buf1, sem1)
            pltpu.make_async_copy(g_hbm.at[idx_r.at[j0]], buf0, sem0).wait()
            pltpu.sync_copy(buf0, acc.at[idx_c.at[j0]], add=True)

            @pl.when(j2 < sz)
            def _():
                pltpu.async_copy(g_hbm.at[idx_r.at[j2]], buf0, sem0)

            pltpu.make_async_copy(g_hbm.at[idx_r.at[j1]], buf1, sem1).wait()
            pltpu.sync_copy(buf1, acc.at[idx_c.at[j1]], add=True)
            return carry
        return body

    base = 0
    for sz in PASS_SZ:
        pltpu.sync_copy(row_hbm.at[w, pl.ds(base, sz)], idx_r.at[pl.ds(0, sz)])
        pltpu.sync_copy(col_hbm.at[w, pl.ds(base, sz)], idx_c.at[pl.ds(0, sz)])
        pltpu.async_copy(g_hbm.at[idx_r.at[0]], buf0, sem0)
        lax.fori_loop(0, sz // 2, make_body(sz), 0)
        base += sz

    plsc.subcore_barrier()
    pltpu.sync_copy(acc.at[pl.ds(r0, ART)], out.at[c, pl.ds(r0, ART)])


_hop = pl.kernel(
    _hop_body,
    out_type=jax.ShapeDtypeStruct((NC, N_PAD, D), jnp.float32),
    mesh=_MESH,
    scratch_types=[
        pltpu.VMEM_SHARED((ACC_R, D), jnp.float32),
        pltpu.VMEM((PASS_SZ[0], CHUNK), jnp.int32),
        pltpu.VMEM((PASS_SZ[0], CHUNK), jnp.int32),
        pltpu.VMEM((CHUNK, D), jnp.float32),
        pltpu.VMEM((CHUNK, D), jnp.float32),
        pltpu.SemaphoreType.DMA,
        pltpu.SemaphoreType.DMA,
    ],
)


def _dinv_block(dg_ref):
    deg = dg_ref[0, :, 0] + dg_ref[1, :, 0] + 1.0
    return 1.0 / jnp.sqrt(deg)


def _scale0_body(dg_ref, x_ref, o_ref, oh_ref):
    dinv = _dinv_block(dg_ref)
    v = x_ref[...] * dinv[:, None]
    o_ref[...] = v
    oh_ref[...] = 0.5 * v


def _mid_body(dg_ref, p_ref, o_ref, oh_ref):
    dinv = _dinv_block(dg_ref)
    v = (p_ref[0] + p_ref[1]) * (dinv * dinv)[:, None]
    o_ref[...] = v
    oh_ref[...] = 0.5 * v


def _final_body(dg_ref, p_ref, w_ref, b_ref, o_ref):
    dinv = _dinv_block(dg_ref)
    h = (p_ref[0] + p_ref[1]) * dinv[:, None]
    z = jnp.dot(h, w_ref[...], preferred_element_type=jnp.float32) + b_ref[...]
    m = jnp.max(z, axis=1, keepdims=True)
    zz = z - m
    lse = jnp.log(jnp.sum(jnp.exp(zz), axis=1, keepdims=True))
    o_ref[...] = zz - lse


_BR = 576
_deg_spec = pl.BlockSpec((NC, _BR, D), lambda i: (0, i, 0))
_row_spec = pl.BlockSpec((_BR, D), lambda i: (i, 0))
_p_spec = pl.BlockSpec((NC, _BR, D), lambda i: (0, i, 0))

_scale0 = pl.pallas_call(
    _scale0_body,
    grid=(N_PAD // _BR,),
    in_specs=[_deg_spec, _row_spec],
    out_specs=(_row_spec, _row_spec),
    out_shape=(jax.ShapeDtypeStruct((N_PAD, D), jnp.float32),
               jax.ShapeDtypeStruct((N_PAD, D), jnp.float32)),
)

_mid = pl.pallas_call(
    _mid_body,
    grid=(N_PAD // _BR,),
    in_specs=[_deg_spec, _p_spec],
    out_specs=(_row_spec, _row_spec),
    out_shape=(jax.ShapeDtypeStruct((N_PAD, D), jnp.float32),
               jax.ShapeDtypeStruct((N_PAD, D), jnp.float32)),
)

_BF = 400
_final = pl.pallas_call(
    _final_body,
    grid=(N // _BF,),
    in_specs=[
        pl.BlockSpec((NC, _BF, D), lambda i: (0, i, 0)),
        pl.BlockSpec((NC, _BF, D), lambda i: (0, i, 0)),
        pl.BlockSpec((D, D), lambda i: (0, 0)),
        pl.BlockSpec((1, D), lambda i: (0, 0)),
    ],
    out_specs=pl.BlockSpec((_BF, D), lambda i: (i, 0)),
    out_shape=jax.ShapeDtypeStruct((N, D), jnp.float32),
)


def kernel(x, edge_index, W, b):
    padn = E_PAD - E
    pad_seq = jnp.arange(padn, dtype=jnp.int32)
    row_pad = N + pad_seq % (ACC_R - N)
    colh_pad = pad_seq % ACC_R
    cold_pad = N + pad_seq % (N_PAD - N)

    row3 = jnp.concatenate([edge_index[0], row_pad]).reshape(NW, CPT, CHUNK)
    colh3 = jnp.concatenate([edge_index[1], colh_pad]).reshape(NW, CPT, CHUNK)
    cold3 = jnp.concatenate([edge_index[1], cold_pad]).reshape(NW, CPT, CHUNK)
    x_pad = jnp.pad(x, ((0, N_PAD - N), (0, 0)))
    z128 = jnp.zeros((N_PAD, D), jnp.float32)
    ones128 = jnp.ones((CHUNK, D), jnp.float32)

    dg = _hist(cold3, z128, ones128)
    g0, g0h = _scale0(dg, x_pad)
    p1 = _hop(g0, row3, colh3, g0h)
    g1, g1h = _mid(dg, p1)
    p2 = _hop(g1, row3, colh3, g1h)
    return _final(dg, p2, W, b.reshape(1, D))
